# Initial kernel scaffold; baseline (speedup 1.0000x reference)
import jax
import jax.numpy as jnp
from jax import lax
from jax.experimental import pallas as pl
from jax.experimental.pallas import tpu as pltpu
from jax.experimental.pallas import tpu_sc as plsc

NUM_ROWS = 100000
DIM = 64
TOTAL = 4096 * 50
NC = 2
NS = 16
NW = NC * NS
PER_W = TOTAL // NW
CHUNK = 128
N_CHUNKS = PER_W // CHUNK


def _emb_body(x_hbm, w_hbm, out_hbm, idx_v, rows_v, sem):
    wid = lax.axis_index("s") * NC + lax.axis_index("c")
    base = wid * PER_W
    pltpu.sync_copy(x_hbm.at[pl.ds(base, PER_W)], idx_v)

    def chunk(g, _):
        pltpu.async_copy(
            w_hbm.at[idx_v.at[pl.ds(g * CHUNK, CHUNK)]], rows_v, sem
        ).wait()
        pltpu.sync_copy(rows_v, out_hbm.at[pl.ds(base + g * CHUNK, CHUNK)])
        return 0

    lax.fori_loop(0, N_CHUNKS, chunk, 0)


@jax.jit
def _emb(x_flat, weight):
    mesh = plsc.VectorSubcoreMesh(
        core_axis_name="c", subcore_axis_name="s", num_cores=NC, num_subcores=NS
    )
    run = pl.kernel(
        _emb_body,
        out_type=jax.ShapeDtypeStruct((TOTAL, DIM), jnp.float32),
        mesh=mesh,
        scratch_types=[
            pltpu.VMEM((PER_W,), jnp.int32),
            pltpu.VMEM((CHUNK, DIM), jnp.float32),
            pltpu.SemaphoreType.DMA,
        ],
    )
    return run(x_flat, weight)


def kernel(x, weight):
    out = _emb(x.reshape(-1), weight)
    return out.reshape(x.shape[0], x.shape[1], DIM)


# speedup vs baseline: 4.0797x; 4.0797x over previous
import jax
import jax.numpy as jnp
from jax import lax
from jax.experimental import pallas as pl
from jax.experimental.pallas import tpu as pltpu
from jax.experimental.pallas import tpu_sc as plsc

NUM_ROWS = 100000
DIM = 64
TOTAL = 4096 * 50
NC = 2
NS = 16
NW = NC * NS
PER_W = TOTAL // NW
CHUNK = 128
N_CHUNKS = PER_W // CHUNK


def _emb_body(x_hbm, w_hbm, out_hbm, idx_v, rows_v, sem):
    wid = lax.axis_index("s") * NC + lax.axis_index("c")
    base = wid * PER_W
    pltpu.sync_copy(x_hbm.at[pl.ds(base, PER_W)], idx_v)

    def chunk(g, _):
        pltpu.async_copy(
            w_hbm.at[idx_v.at[pl.ds(g * CHUNK, CHUNK)]], rows_v, sem
        ).wait()
        pltpu.sync_copy(rows_v, out_hbm.at[pl.ds(base + g * CHUNK, CHUNK)])
        return 0

    lax.fori_loop(0, N_CHUNKS, chunk, 0)


@jax.jit
def _emb(x_flat, weight):
    mesh = plsc.VectorSubcoreMesh(
        core_axis_name="c", subcore_axis_name="s", num_cores=NC, num_subcores=NS
    )
    run = pl.kernel(
        _emb_body,
        out_type=jax.ShapeDtypeStruct((TOTAL, DIM), jnp.float32),
        mesh=mesh,
        scratch_types=[
            pltpu.VMEM((PER_W,), jnp.int32),
            pltpu.VMEM((CHUNK, DIM), jnp.float32),
            pltpu.SemaphoreType.DMA,
        ],
        compiler_params=pltpu.CompilerParams(use_tc_tiling_on_sc=False),
    )
    return run(x_flat, weight)


def kernel(x, weight):
    out = _emb(x.reshape(-1), weight)
    return out.reshape(x.shape[0], x.shape[1], DIM)


# speedup vs baseline: 4.6661x; 1.1437x over previous
import jax
import jax.numpy as jnp
from jax import lax
from jax.experimental import pallas as pl
from jax.experimental.pallas import tpu as pltpu
from jax.experimental.pallas import tpu_sc as plsc

NUM_ROWS = 100000
DIM = 64
TOTAL = 4096 * 50
NC = 2
NS = 16
NW = NC * NS
PER_W = TOTAL // NW
CHUNK = 128
GROUP = 5
NBUF = 3
GROUP_ROWS = GROUP * CHUNK
N_GROUPS = PER_W // GROUP_ROWS


def _emb_body(x_hbm, w_hbm, out_hbm, idx_v, rows_v, *sems):
    gsem = sems[:NBUF]
    wsem = sems[NBUF:]
    wid = lax.axis_index("s") * NC + lax.axis_index("c")
    base = wid * PER_W
    pltpu.sync_copy(x_hbm.at[pl.ds(base, PER_W)], idx_v)

    gather_descs = {}
    write_descs = {}

    def fire_gathers(G):
        b = G % NBUF
        descs = []
        for j in range(GROUP):
            g = G * GROUP + j
            descs.append(
                pltpu.async_copy(
                    w_hbm.at[idx_v.at[pl.ds(g * CHUNK, CHUNK)]],
                    rows_v.at[b, pl.ds(j * CHUNK, CHUNK)],
                    gsem[b],
                )
            )
        gather_descs[G] = descs

    fire_gathers(0)
    fire_gathers(1)
    for G in range(N_GROUPS):
        H = G + 2
        if H < N_GROUPS:
            if H - NBUF >= 0:
                write_descs[H - NBUF].wait()
            fire_gathers(H)
        for d in gather_descs[G]:
            d.wait()
        write_descs[G] = pltpu.async_copy(
            rows_v.at[G % NBUF],
            out_hbm.at[pl.ds(base + G * GROUP_ROWS, GROUP_ROWS)],
            wsem[G % NBUF],
        )
    for G in range(N_GROUPS - NBUF, N_GROUPS):
        write_descs[G].wait()


@jax.jit
def _emb(x_flat, weight):
    mesh = plsc.VectorSubcoreMesh(
        core_axis_name="c", subcore_axis_name="s", num_cores=NC, num_subcores=NS
    )
    run = pl.kernel(
        _emb_body,
        out_type=jax.ShapeDtypeStruct((TOTAL, DIM), jnp.float32),
        mesh=mesh,
        scratch_types=[
            pltpu.VMEM((PER_W,), jnp.int32),
            pltpu.VMEM((NBUF, GROUP_ROWS, DIM), jnp.float32),
        ]
        + [pltpu.SemaphoreType.DMA] * (2 * NBUF),
        compiler_params=pltpu.CompilerParams(use_tc_tiling_on_sc=False),
    )
    return run(x_flat, weight)


def kernel(x, weight):
    out = _emb(x.reshape(-1), weight)
    return out.reshape(x.shape[0], x.shape[1], DIM)
